# baseline (speedup 1.0000x reference)
import functools

import jax
import jax.numpy as jnp
from jax import lax
from jax.experimental import pallas as pl
from jax.experimental.pallas import tpu as pltpu
from jax.experimental.pallas import tpu_sc as plsc

_N = 10000
_K = 32
_D = 128

_NW = 32
_NSC = 5120
_NTC = _N - _NSC
_NPW = _NSC // _NW
_CB = 4
_IDXC = _CB * _K
_CPW = _NPW // _CB
_LANES = 16
_NV = _D // _LANES


def _sc_neighbor_sum(x, edge_groups):
    mesh = plsc.VectorSubcoreMesh(core_axis_name="c", subcore_axis_name="s")

    @functools.partial(
        pl.kernel,
        out_type=jax.ShapeDtypeStruct((_NSC, _D), jnp.float32),
        mesh=mesh,
        scratch_types=[
            pltpu.VMEM((_CPW, _IDXC), jnp.int32),
            pltpu.VMEM((_IDXC, _D), jnp.float32),
            pltpu.VMEM((_IDXC, _D), jnp.float32),
            pltpu.VMEM((_NPW, _D), jnp.float32),
            pltpu.SemaphoreType.DMA,
            pltpu.SemaphoreType.DMA,
        ],
    )
    def body(x_hbm, edge_hbm, out_hbm, idx_v, rows0, rows1, out_v, sem0, sem1):
        wid = lax.axis_index("s") * 2 + lax.axis_index("c")

        pltpu.sync_copy(edge_hbm.at[wid], idx_v)

        def start(g, rows_ref, sem):
            pltpu.async_copy(x_hbm.at[idx_v.at[g]], rows_ref, sem)

        def wait(rows_ref, sem):
            pltpu.make_async_copy(x_hbm.at[pl.ds(0, _IDXC)], rows_ref, sem).wait()

        def compute(g, rows_ref):
            base_slot = g * _CB
            for b in range(_CB):
                def kstep(k, accs, _b=b):
                    r = _b * _K + 2 * k
                    accs = tuple(
                        accs[c] + rows_ref[r, pl.ds(c * _LANES, _LANES)]
                        for c in range(_NV)
                    )
                    return tuple(
                        accs[c] + rows_ref[r + 1, pl.ds(c * _LANES, _LANES)]
                        for c in range(_NV)
                    )

                zeros = tuple(jnp.zeros((_LANES,), jnp.float32) for _ in range(_NV))
                accs = lax.fori_loop(0, _K // 2, kstep, zeros)
                for c in range(_NV):
                    out_v[base_slot + b, pl.ds(c * _LANES, _LANES)] = accs[c]

        start(0, rows0, sem0)
        start(1, rows1, sem1)

        def pair_body(p, carry):
            g = 2 * p
            wait(rows0, sem0)
            compute(g, rows0)

            @pl.when(p + 1 < _CPW // 2)
            def _():
                start(g + 2, rows0, sem0)

            wait(rows1, sem1)
            compute(g + 1, rows1)

            @pl.when(p + 1 < _CPW // 2)
            def _():
                start(g + 3, rows1, sem1)

            return carry

        lax.fori_loop(0, _CPW // 2, pair_body, 0)

        pltpu.sync_copy(out_v, out_hbm.at[pl.ds(wid * _NPW, _NPW)])

    return body(x, edge_groups)


def _tc_lin_body(x_ref, ns_ref, eps_ref, wt_ref, b_ref, o_ref):
    h = (1.0 + eps_ref[0, 0]) * x_ref[...] + ns_ref[...]
    o_ref[...] = (
        jnp.dot(h, wt_ref[...], preferred_element_type=jnp.float32) + b_ref[...]
    )


def _tc_linear(xs, nsum, eps11, wt, b1):
    n, br = _NSC, 640
    return pl.pallas_call(
        _tc_lin_body,
        grid=(n // br,),
        in_specs=[
            pl.BlockSpec((br, _D), lambda i: (i, 0)),
            pl.BlockSpec((br, _D), lambda i: (i, 0)),
            pl.BlockSpec(memory_space=pltpu.SMEM),
            pl.BlockSpec((_D, _D), lambda i: (0, 0)),
            pl.BlockSpec((1, _D), lambda i: (0, 0)),
        ],
        out_specs=pl.BlockSpec((br, _D), lambda i: (i, 0)),
        out_shape=jax.ShapeDtypeStruct((n, _D), jnp.float32),
    )(xs, nsum, eps11, wt, b1)


_BRT = 488


def _tc_gather_body(x_all_ref, xb_ref, e_ref, eps_ref, wt_ref, b_ref,
                    o_ref, ns_ref):
    def node_body(i, carry):
        acc0 = x_all_ref[pl.ds(e_ref[i, 0], 1), :]
        acc1 = x_all_ref[pl.ds(e_ref[i, 1], 1), :]
        acc2 = x_all_ref[pl.ds(e_ref[i, 2], 1), :]
        acc3 = x_all_ref[pl.ds(e_ref[i, 3], 1), :]
        for k in range(4, _K, 4):
            acc0 = acc0 + x_all_ref[pl.ds(e_ref[i, k], 1), :]
            acc1 = acc1 + x_all_ref[pl.ds(e_ref[i, k + 1], 1), :]
            acc2 = acc2 + x_all_ref[pl.ds(e_ref[i, k + 2], 1), :]
            acc3 = acc3 + x_all_ref[pl.ds(e_ref[i, k + 3], 1), :]
        ns_ref[pl.ds(i, 1), :] = (acc0 + acc1) + (acc2 + acc3)
        return carry

    lax.fori_loop(0, _BRT, node_body, 0)
    h = (1.0 + eps_ref[0, 0]) * xb_ref[...] + ns_ref[...]
    o_ref[...] = (
        jnp.dot(h, wt_ref[...], preferred_element_type=jnp.float32) + b_ref[...]
    )


def _tc_gather_linear(x, xt, edge_t, eps11, wt, b1):
    return pl.pallas_call(
        _tc_gather_body,
        grid=(_NTC // _BRT,),
        in_specs=[
            pl.BlockSpec((_N, _D), lambda i: (0, 0)),
            pl.BlockSpec((_BRT, _D), lambda i: (i, 0)),
            pl.BlockSpec((_BRT, _K), lambda i: (i, 0), memory_space=pltpu.SMEM),
            pl.BlockSpec(memory_space=pltpu.SMEM),
            pl.BlockSpec((_D, _D), lambda i: (0, 0)),
            pl.BlockSpec((1, _D), lambda i: (0, 0)),
        ],
        out_specs=pl.BlockSpec((_BRT, _D), lambda i: (i, 0)),
        out_shape=jax.ShapeDtypeStruct((_NTC, _D), jnp.float32),
        scratch_shapes=[pltpu.VMEM((_BRT, _D), jnp.float32)],
    )(x, xt, edge_t, eps11, wt, b1)


def kernel(x, edge_index, eps, W, b):
    edge_groups = edge_index[:_NSC].reshape(_NW, _CPW, _IDXC)
    nsum_sc = _sc_neighbor_sum(x, edge_groups)
    eps11 = eps.reshape(1, 1)
    wt = W.T
    b1 = b.reshape(1, _D)
    out_tc = _tc_gather_linear(x, x[_NSC:], edge_index[_NSC:], eps11, wt, b1)
    out_sc = _tc_linear(x[:_NSC], nsum_sc, eps11, wt, b1)
    return jnp.concatenate([out_sc, out_tc], axis=0)


# speedup vs baseline: 1.6100x; 1.6100x over previous
import functools

import jax
import jax.numpy as jnp
from jax import lax
from jax.experimental import pallas as pl
from jax.experimental.pallas import tpu as pltpu
from jax.experimental.pallas import tpu_sc as plsc

_N = 10000
_K = 32
_D = 128

_NW = 32
_NSC = 7680
_NTC = _N - _NSC
_NPW = _NSC // _NW
_CB = 4
_IDXC = _CB * _K
_CPW = _NPW // _CB
_LANES = 16
_NV = _D // _LANES


def _sc_neighbor_sum(x, edge_groups):
    mesh = plsc.VectorSubcoreMesh(core_axis_name="c", subcore_axis_name="s")

    @functools.partial(
        pl.kernel,
        out_type=jax.ShapeDtypeStruct((_NSC, _D), jnp.float32),
        mesh=mesh,
        scratch_types=[
            pltpu.VMEM((_CPW, _IDXC), jnp.int32),
            pltpu.VMEM((_IDXC, _D), jnp.float32),
            pltpu.VMEM((_IDXC, _D), jnp.float32),
            pltpu.VMEM((_NPW, _D), jnp.float32),
            pltpu.SemaphoreType.DMA,
            pltpu.SemaphoreType.DMA,
        ],
    )
    def body(x_hbm, edge_hbm, out_hbm, idx_v, rows0, rows1, out_v, sem0, sem1):
        wid = lax.axis_index("s") * 2 + lax.axis_index("c")

        pltpu.sync_copy(edge_hbm.at[wid], idx_v)

        def start(g, rows_ref, sem):
            pltpu.async_copy(x_hbm.at[idx_v.at[g]], rows_ref, sem)

        def wait(rows_ref, sem):
            pltpu.make_async_copy(x_hbm.at[pl.ds(0, _IDXC)], rows_ref, sem).wait()

        def compute(g, rows_ref):
            base_slot = g * _CB
            for b in range(_CB):
                def kstep(k, accs, _b=b):
                    r = _b * _K + 2 * k
                    accs = tuple(
                        accs[c] + rows_ref[r, pl.ds(c * _LANES, _LANES)]
                        for c in range(_NV)
                    )
                    return tuple(
                        accs[c] + rows_ref[r + 1, pl.ds(c * _LANES, _LANES)]
                        for c in range(_NV)
                    )

                zeros = tuple(jnp.zeros((_LANES,), jnp.float32) for _ in range(_NV))
                accs = lax.fori_loop(0, _K // 2, kstep, zeros)
                for c in range(_NV):
                    out_v[base_slot + b, pl.ds(c * _LANES, _LANES)] = accs[c]

        start(0, rows0, sem0)
        start(1, rows1, sem1)

        def pair_body(p, carry):
            g = 2 * p
            wait(rows0, sem0)
            compute(g, rows0)

            @pl.when(p + 1 < _CPW // 2)
            def _():
                start(g + 2, rows0, sem0)

            wait(rows1, sem1)
            compute(g + 1, rows1)

            @pl.when(p + 1 < _CPW // 2)
            def _():
                start(g + 3, rows1, sem1)

            return carry

        lax.fori_loop(0, _CPW // 2, pair_body, 0)

        pltpu.sync_copy(out_v, out_hbm.at[pl.ds(wid * _NPW, _NPW)])

    return body(x, edge_groups)


def _tc_lin_body(x_ref, ns_ref, eps_ref, wt_ref, b_ref, o_ref):
    h = (1.0 + eps_ref[0, 0]) * x_ref[...] + ns_ref[...]
    o_ref[...] = (
        jnp.dot(h, wt_ref[...], preferred_element_type=jnp.float32) + b_ref[...]
    )


def _tc_linear(xs, nsum, eps11, wt, b1):
    n, br = _NSC, 640
    return pl.pallas_call(
        _tc_lin_body,
        grid=(n // br,),
        in_specs=[
            pl.BlockSpec((br, _D), lambda i: (i, 0)),
            pl.BlockSpec((br, _D), lambda i: (i, 0)),
            pl.BlockSpec(memory_space=pltpu.SMEM),
            pl.BlockSpec((_D, _D), lambda i: (0, 0)),
            pl.BlockSpec((1, _D), lambda i: (0, 0)),
        ],
        out_specs=pl.BlockSpec((br, _D), lambda i: (i, 0)),
        out_shape=jax.ShapeDtypeStruct((n, _D), jnp.float32),
    )(xs, nsum, eps11, wt, b1)


_BRT = 80


def _tc_gather_body(x_all_ref, xb_ref, e_ref, eps_ref, wt_ref, b_ref,
                    o_ref, ns_ref):
    def node_body(i, carry):
        acc0 = x_all_ref[pl.ds(e_ref[i, 0], 1), :]
        acc1 = x_all_ref[pl.ds(e_ref[i, 1], 1), :]
        acc2 = x_all_ref[pl.ds(e_ref[i, 2], 1), :]
        acc3 = x_all_ref[pl.ds(e_ref[i, 3], 1), :]
        for k in range(4, _K, 4):
            acc0 = acc0 + x_all_ref[pl.ds(e_ref[i, k], 1), :]
            acc1 = acc1 + x_all_ref[pl.ds(e_ref[i, k + 1], 1), :]
            acc2 = acc2 + x_all_ref[pl.ds(e_ref[i, k + 2], 1), :]
            acc3 = acc3 + x_all_ref[pl.ds(e_ref[i, k + 3], 1), :]
        ns_ref[pl.ds(i, 1), :] = (acc0 + acc1) + (acc2 + acc3)
        return carry

    lax.fori_loop(0, _BRT, node_body, 0)
    h = (1.0 + eps_ref[0, 0]) * xb_ref[...] + ns_ref[...]
    o_ref[...] = (
        jnp.dot(h, wt_ref[...], preferred_element_type=jnp.float32) + b_ref[...]
    )


def _tc_gather_linear(x, xt, edge_t, eps11, wt, b1):
    return pl.pallas_call(
        _tc_gather_body,
        grid=(_NTC // _BRT,),
        in_specs=[
            pl.BlockSpec((_N, _D), lambda i: (0, 0)),
            pl.BlockSpec((_BRT, _D), lambda i: (i, 0)),
            pl.BlockSpec((_BRT, _K), lambda i: (i, 0), memory_space=pltpu.SMEM),
            pl.BlockSpec(memory_space=pltpu.SMEM),
            pl.BlockSpec((_D, _D), lambda i: (0, 0)),
            pl.BlockSpec((1, _D), lambda i: (0, 0)),
        ],
        out_specs=pl.BlockSpec((_BRT, _D), lambda i: (i, 0)),
        out_shape=jax.ShapeDtypeStruct((_NTC, _D), jnp.float32),
        scratch_shapes=[pltpu.VMEM((_BRT, _D), jnp.float32)],
    )(x, xt, edge_t, eps11, wt, b1)


def kernel(x, edge_index, eps, W, b):
    edge_groups = edge_index[:_NSC].reshape(_NW, _CPW, _IDXC)
    nsum_sc = _sc_neighbor_sum(x, edge_groups)
    eps11 = eps.reshape(1, 1)
    wt = W.T
    b1 = b.reshape(1, _D)
    out_tc = _tc_gather_linear(x, x[_NSC:], edge_index[_NSC:], eps11, wt, b1)
    out_sc = _tc_linear(x[:_NSC], nsum_sc, eps11, wt, b1)
    return jnp.concatenate([out_sc, out_tc], axis=0)
